# baseline (speedup 1.0000x reference)
import functools
import math

import numpy as np
import jax
import jax.numpy as jnp
from jax import lax
from jax.experimental import pallas as pl
from jax.experimental.pallas import tpu as pltpu
from jax.experimental.pallas import tpu_sc as plsc

N_VOCAB = 100000
MAX_LENGTH = 4096
OUT_DIM = 128


def _make_pe(max_length, out_dim):
    position = np.arange(max_length, dtype=np.float32)[:, None]
    div_term = np.exp(
        np.arange(0, out_dim, 2, dtype=np.float32) * -(math.log(10000.0) / out_dim)
    )
    pe = np.zeros((max_length, out_dim), dtype=np.float32)
    pe[:, 0::2] = np.sin(position * div_term)
    pe[:, 1::2] = np.cos(position * div_term)
    return pe


def _make_consts(seq, dim, nw):
    pe = _make_pe(MAX_LENGTH, dim)
    ppw = seq // nw
    div_term = np.exp(
        np.arange(0, dim, 2, dtype=np.float64) * -(math.log(10000.0) / dim)
    )
    cc = np.repeat(np.cos(div_term), 2)
    ss = np.empty(dim, np.float64)
    ss[0::2] = np.sin(div_term)
    ss[1::2] = -np.sin(div_term)
    consts = np.zeros((2 * nw + 2, dim), np.float32)
    for w in range(nw):
        seed = pe[w * ppw]
        consts[w] = seed
        consts[nw + w] = seed.reshape(-1, 2)[:, ::-1].reshape(-1)
    consts[2 * nw] = cc
    consts[2 * nw + 1] = ss
    return consts


@functools.cache
def _build(batch, seq, dim):
    info = plsc.get_sparse_core_info()
    nc, ns, lanes = info.num_cores, info.num_subcores, info.num_lanes
    nw = nc * ns
    assert seq % nw == 0 and dim % lanes == 0
    ppw = seq // nw
    n_chunks = dim // lanes

    mesh = plsc.VectorSubcoreMesh(core_axis_name="c", subcore_axis_name="s")

    @functools.partial(
        pl.kernel,
        mesh=mesh,
        out_type=jax.ShapeDtypeStruct((batch, seq, dim), jnp.float32),
        scratch_types=[
            pltpu.VMEM((batch, ppw), jnp.int32),
            pltpu.VMEM((4, dim), jnp.float32),
            pltpu.VMEM((ppw, dim), jnp.float32),
            pltpu.VMEM((2, ppw, dim), jnp.float32),
            pltpu.SemaphoreType.DMA,
            pltpu.SemaphoreType.DMA,
        ],
    )
    def emb(idx_hbm, table_hbm, consts_hbm, out_hbm,
            idx_v, aux_v, pe_v, rows_v, gsem, ssem):
        wid = lax.axis_index("s") * nc + lax.axis_index("c")
        pos0 = wid * ppw

        for b in range(batch):
            pltpu.sync_copy(idx_hbm.at[b, pl.ds(pos0, ppw)], idx_v.at[b])

        gathers = [None] * batch
        gathers[0] = pltpu.async_copy(
            table_hbm.at[idx_v.at[0]], rows_v.at[0], gsem
        )

        pltpu.sync_copy(consts_hbm.at[wid], aux_v.at[0])
        pltpu.sync_copy(consts_hbm.at[nw + wid], aux_v.at[1])
        pltpu.sync_copy(consts_hbm.at[2 * nw], aux_v.at[2])
        pltpu.sync_copy(consts_hbm.at[2 * nw + 1], aux_v.at[3])

        def chunk(c):
            return pl.ds(c * lanes, lanes)

        ccs = [aux_v[2, chunk(c)] for c in range(n_chunks)]
        sss = [aux_v[3, chunk(c)] for c in range(n_chunks)]
        xs0 = [aux_v[0, chunk(c)] for c in range(n_chunks)]
        ys0 = [aux_v[1, chunk(c)] for c in range(n_chunks)]
        for c in range(n_chunks):
            pe_v[0, chunk(c)] = xs0[c]

        def pe_row(r, carry):
            xs, ys = carry
            nxs, nys = [], []
            for c in range(n_chunks):
                nx = xs[c] * ccs[c] + ys[c] * sss[c]
                ny = ys[c] * ccs[c] - xs[c] * sss[c]
                pe_v[r, chunk(c)] = nx
                nxs.append(nx)
                nys.append(ny)
            return (tuple(nxs), tuple(nys))

        lax.fori_loop(1, ppw, pe_row, (tuple(xs0), tuple(ys0)))

        stores = [None] * batch
        for b in range(batch):
            buf = b % 2
            if b + 1 < batch:
                if stores[b - 1] is not None:
                    stores[b - 1].wait()
                gathers[b + 1] = pltpu.async_copy(
                    table_hbm.at[idx_v.at[b + 1]], rows_v.at[(b + 1) % 2], gsem
                )
            gathers[b].wait()

            def row_add(r, carry, buf=buf):
                for c in range(n_chunks):
                    sl = pl.ds(c * lanes, lanes)
                    rows_v[buf, r, sl] = rows_v[buf, r, sl] + pe_v[r, sl]
                return carry

            lax.fori_loop(0, ppw, row_add, 0)

            stores[b] = pltpu.async_copy(
                rows_v.at[buf], out_hbm.at[b, pl.ds(pos0, ppw)], ssem
            )
        stores[batch - 2].wait()
        stores[batch - 1].wait()

    return emb, nw


def kernel(input_ids, table):
    batch, seq = input_ids.shape
    dim = table.shape[1]
    idx = input_ids.astype(jnp.int32)
    emb, nw = _build(batch, seq, dim)
    consts = jnp.asarray(_make_consts(seq, dim, nw))
    return emb(idx, table, consts)
